# Initial kernel scaffold; baseline (speedup 1.0000x reference)
import jax
import jax.numpy as jnp
from jax.experimental import pallas as pl


def kernel(x, edge_index):
    raise NotImplementedError("write your pallas kernel here")


# speedup vs baseline: 2.6006x; 2.6006x over previous
import functools

import jax
import jax.numpy as jnp
from jax import lax
from jax.experimental import pallas as pl
from jax.experimental.pallas import tpu as pltpu
from jax.experimental.pallas import tpu_sc as plsc

D = 128
C = 80
L = 16


def _sc_scores(x, src, dst):
    E = src.shape[0]
    info = plsc.get_sparse_core_info()
    NW = info.num_cores * info.num_subcores
    per_w = E // NW
    n_chunks = per_w // C
    mesh = plsc.VectorSubcoreMesh(core_axis_name="c", subcore_axis_name="s")

    @functools.partial(
        pl.kernel,
        mesh=mesh,
        compiler_params=pltpu.CompilerParams(needs_layout_passes=False),
        out_type=jax.ShapeDtypeStruct((E,), jnp.float32),
        scratch_types=[
            pltpu.VMEM((C,), jnp.int32),
            pltpu.VMEM((C,), jnp.int32),
            pltpu.VMEM((C, D), jnp.float32),
            pltpu.VMEM((C, D), jnp.float32),
            pltpu.VMEM((C,), jnp.float32),
            pltpu.SemaphoreType.DMA,
        ],
    )
    def k(x_hbm, src_hbm, dst_hbm, out_hbm, sidx, didx, srows, drows, outv, sem):
        wid = lax.axis_index("s") * info.num_cores + lax.axis_index("c")
        lane = lax.broadcasted_iota(jnp.int32, (L,), 0)

        def chunk_body(ci, carry):
            base = wid * per_w + ci * C
            pltpu.sync_copy(src_hbm.at[pl.ds(base, C)], sidx)
            pltpu.sync_copy(dst_hbm.at[pl.ds(base, C)], didx)
            cp_s = pltpu.async_copy(x_hbm.at[sidx], srows, sem)
            cp_d = pltpu.async_copy(x_hbm.at[didx], drows, sem)
            cp_s.wait()
            cp_d.wait()
            for g in range(C // L):
                row = g * L + lane

                def f_body(f, acc):
                    col = (f + lane) & (D - 1)
                    s = plsc.load_gather(srows, [row, col])
                    d_ = plsc.load_gather(drows, [row, col])
                    return acc + s * d_

                acc = lax.fori_loop(0, D, f_body, jnp.zeros((L,), jnp.float32))
                outv[pl.ds(g * L, L)] = acc
            pltpu.sync_copy(outv, out_hbm.at[pl.ds(base, C)])
            return carry

        lax.fori_loop(0, n_chunks, chunk_body, 0)

    return k(x, src, dst)


def kernel(x, edge_index):
    ei = edge_index.astype(jnp.int32)
    scores = _sc_scores(x, ei[0], ei[1])
    return scores.reshape(-1, 1)


# speedup vs baseline: 8.8188x; 3.3911x over previous
import functools

import jax
import jax.numpy as jnp
from jax import lax
from jax.experimental import pallas as pl
from jax.experimental.pallas import tpu as pltpu
from jax.experimental.pallas import tpu_sc as plsc

D = 128
C = 80
L = 16


def _sc_scores(x, src, dst):
    E = src.shape[0]
    info = plsc.get_sparse_core_info()
    NW = info.num_cores * info.num_subcores
    per_w = E // NW
    n_chunks = per_w // C
    n_pairs = (n_chunks - 1) // 2
    mesh = plsc.VectorSubcoreMesh(core_axis_name="c", subcore_axis_name="s")

    @functools.partial(
        pl.kernel,
        mesh=mesh,
        compiler_params=pltpu.CompilerParams(needs_layout_passes=False),
        out_type=jax.ShapeDtypeStruct((E,), jnp.float32),
        scratch_types=[
            pltpu.VMEM((per_w,), jnp.int32),
            pltpu.VMEM((per_w,), jnp.int32),
            pltpu.VMEM((C, D), jnp.float32),
            pltpu.VMEM((C, D), jnp.float32),
            pltpu.VMEM((C, D), jnp.float32),
            pltpu.VMEM((C, D), jnp.float32),
            pltpu.VMEM((per_w,), jnp.float32),
            pltpu.SemaphoreType.DMA,
            pltpu.SemaphoreType.DMA,
        ],
    )
    def k(x_hbm, src_hbm, dst_hbm, out_hbm,
          sidx, didx, sr0, dr0, sr1, dr1, outv, sem0, sem1):
        wid = lax.axis_index("s") * info.num_cores + lax.axis_index("c")
        base = wid * per_w
        lane = lax.broadcasted_iota(jnp.int32, (L,), 0)
        bufs = ((sr0, dr0, sem0), (sr1, dr1, sem1))

        pltpu.sync_copy(src_hbm.at[pl.ds(base, per_w)], sidx)
        pltpu.sync_copy(dst_hbm.at[pl.ds(base, per_w)], didx)

        def fire(ci, b):
            sr, dr, sem = bufs[b]
            pltpu.async_copy(x_hbm.at[sidx.at[pl.ds(ci * C, C)]], sr, sem)
            pltpu.async_copy(x_hbm.at[didx.at[pl.ds(ci * C, C)]], dr, sem)

        def wait(b):
            sr, dr, sem = bufs[b]
            pltpu.make_async_copy(x_hbm.at[sidx.at[pl.ds(0, C)]], sr, sem).wait()
            pltpu.make_async_copy(x_hbm.at[didx.at[pl.ds(0, C)]], dr, sem).wait()

        def compute(ci, b):
            sr, dr, _ = bufs[b]
            for g in range(C // L):
                row = g * L + lane

                def f_body(f, acc):
                    col = (f + lane) & (D - 1)
                    s = plsc.load_gather(sr, [row, col])
                    d_ = plsc.load_gather(dr, [row, col])
                    return acc + s * d_

                acc = lax.fori_loop(0, D, f_body, jnp.zeros((L,), jnp.float32),
                                    unroll=4)
                outv[pl.ds(ci * C + g * L, L)] = acc

        fire(0, 0)

        def pair_body(s, carry):
            ci0 = 2 * s
            fire(ci0 + 1, 1)
            wait(0)
            compute(ci0, 0)
            fire(ci0 + 2, 0)
            wait(1)
            compute(ci0 + 1, 1)
            return carry

        lax.fori_loop(0, n_pairs, pair_body, 0)
        wait(0)
        compute(n_chunks - 1, 0)
        pltpu.sync_copy(outv, out_hbm.at[pl.ds(base, per_w)])

    return k(x, src, dst)


def kernel(x, edge_index):
    ei = edge_index.astype(jnp.int32)
    scores = _sc_scores(x, ei[0], ei[1])
    return scores.reshape(-1, 1)
